# baseline (speedup 1.0000x reference)
import functools

import jax
import jax.numpy as jnp
from jax import lax
from jax.experimental import pallas as pl
from jax.experimental.pallas import tpu as pltpu
from jax.experimental.pallas import tpu_sc as plsc

F = 26
V1 = 100001
D = 32
L = 16
NW = 32
OCH = 2048


def _gather_kernel(B, idx_hbm, tab_hbm, out_hbm, idx_v, row_v, ob0, ob1,
                   *sems):
    obufs = (ob0, ob1)
    per_w = (F * D) // NW
    n_och = B // OCH
    wid = lax.axis_index("c") * 16 + lax.axis_index("s")
    p0 = wid * per_w
    for g in range(1, 4):
        @pl.when((wid & 3) == g)
        def _():
            pl.delay(g * 2000)

    def load_idx(f):
        pltpu.sync_copy(idx_hbm.at[f], idx_v)

        def clip16(i, _):
            for u in range(16):
                sl = pl.ds(i * 256 + u * L, L)
                idx_v[sl] = jnp.minimum(jnp.maximum(idx_v[sl], 0), V1 - 1)
            return 0

        lax.fori_loop(0, B // 256, clip16, 0)

    def task(t, _):
        p = p0 + t
        f = lax.shift_right_logical(p, 5)
        d = lax.bitwise_and(p, D - 1)

        @pl.when(jnp.logical_or(t == 0, d == 0))
        def _():
            load_idx(f)

        pltpu.sync_copy(tab_hbm.at[f, d], row_v)

        for c in range(n_och):
            buf = obufs[c % 2]
            prev = pltpu.make_async_copy(
                buf, out_hbm.at[0, 0, pl.ds(0, OCH)], sems[c % 2])
            if c >= 2:
                prev.wait()
            else:
                @pl.when(t > 0)
                def _():
                    prev.wait()

            def g16(i, _):
                for u in range(16):
                    j = i * 16 + u
                    iv = idx_v[pl.ds(c * OCH + j * L, L)]
                    buf[pl.ds(j * L, L)] = plsc.load_gather(row_v, [iv])
                return 0

            lax.fori_loop(0, OCH // (16 * L), g16, 0)
            pltpu.async_copy(
                buf, out_hbm.at[f, d, pl.ds(c * OCH, OCH)], sems[c % 2])
        return 0

    lax.fori_loop(0, per_w, task, 0)
    for k in range(2):
        pltpu.make_async_copy(
            obufs[k], out_hbm.at[0, 0, pl.ds(0, OCH)], sems[k]).wait()


def kernel(int_feats, tables):
    B, nf = int_feats.shape
    assert nf == F and tables.shape == (F, V1, D)
    assert (F * D) % NW == 0 and B % 256 == 0 and B % OCH == 0

    idx_t = int_feats.T
    tab_t = tables.transpose(0, 2, 1)

    mesh = plsc.VectorSubcoreMesh(core_axis_name="c", subcore_axis_name="s",
                                  num_cores=2, num_subcores=16)
    run = pl.kernel(
        functools.partial(_gather_kernel, B),
        out_type=jax.ShapeDtypeStruct((F, D, B), jnp.float32),
        mesh=mesh,
        scratch_types=(
            [pltpu.VMEM((B,), jnp.int32),
             pltpu.VMEM((V1,), jnp.float32),
             pltpu.VMEM((OCH,), jnp.float32),
             pltpu.VMEM((OCH,), jnp.float32)]
            + [pltpu.SemaphoreType.DMA] * 2
        ),
        compiler_params=pltpu.CompilerParams(needs_layout_passes=False),
    )
    out_t = run(idx_t, tab_t)
    return out_t.transpose(2, 0, 1)


# speedup vs baseline: 1.4133x; 1.4133x over previous
import functools

import jax
import jax.numpy as jnp
from jax import lax
from jax.experimental import pallas as pl
from jax.experimental.pallas import tpu as pltpu
from jax.experimental.pallas import tpu_sc as plsc

F = 26
V1 = 100001
D = 32
L = 16
NW = 32
OCH = 2048


def _gather_kernel(B, idx_hbm, tab_hbm, out_hbm, idx_v, row_v, ob0, ob1,
                   *sems):
    obufs = (ob0, ob1)
    per_w = (F * D) // NW
    n_och = B // OCH
    wid = lax.axis_index("c") * 16 + lax.axis_index("s")
    p0 = wid * per_w

    def load_idx(f):
        pltpu.sync_copy(idx_hbm.at[f], idx_v)

        def clip16(i, _):
            for u in range(16):
                sl = pl.ds(i * 256 + u * L, L)
                idx_v[sl] = jnp.minimum(jnp.maximum(idx_v[sl], 0), V1 - 1)
            return 0

        lax.fori_loop(0, B // 256, clip16, 0)

    def task(t, _):
        p = p0 + t
        f = lax.shift_right_logical(p, 5)
        d = lax.bitwise_and(p, D - 1)

        @pl.when(jnp.logical_or(t == 0, d == 0))
        def _():
            load_idx(f)

        pltpu.sync_copy(tab_hbm.at[f, d], row_v)

        for c in range(n_och):
            buf = obufs[c % 2]
            prev = pltpu.make_async_copy(
                buf, out_hbm.at[0, 0, pl.ds(0, OCH)], sems[c % 2])
            if c >= 2:
                prev.wait()
            else:
                @pl.when(t > 0)
                def _():
                    prev.wait()

            @plsc.parallel_loop(0, OCH // L, unroll=8)
            def _(j):
                iv = idx_v[pl.ds(c * OCH + j * L, L)]
                buf[pl.ds(j * L, L)] = plsc.load_gather(row_v, [iv])
            pltpu.async_copy(
                buf, out_hbm.at[f, d, pl.ds(c * OCH, OCH)], sems[c % 2])
        return 0

    lax.fori_loop(0, per_w, task, 0)
    for k in range(2):
        pltpu.make_async_copy(
            obufs[k], out_hbm.at[0, 0, pl.ds(0, OCH)], sems[k]).wait()


def kernel(int_feats, tables):
    B, nf = int_feats.shape
    assert nf == F and tables.shape == (F, V1, D)
    assert (F * D) % NW == 0 and B % 256 == 0 and B % OCH == 0

    idx_t = int_feats.T
    tab_t = tables.transpose(0, 2, 1)

    mesh = plsc.VectorSubcoreMesh(core_axis_name="c", subcore_axis_name="s",
                                  num_cores=2, num_subcores=16)
    run = pl.kernel(
        functools.partial(_gather_kernel, B),
        out_type=jax.ShapeDtypeStruct((F, D, B), jnp.float32),
        mesh=mesh,
        scratch_types=(
            [pltpu.VMEM((B,), jnp.int32),
             pltpu.VMEM((V1,), jnp.float32),
             pltpu.VMEM((OCH,), jnp.float32),
             pltpu.VMEM((OCH,), jnp.float32)]
            + [pltpu.SemaphoreType.DMA] * 2
        ),
        compiler_params=pltpu.CompilerParams(needs_layout_passes=False),
    )
    out_t = run(idx_t, tab_t)
    return out_t.transpose(2, 0, 1)


# speedup vs baseline: 1.7275x; 1.2223x over previous
import functools

import jax
import jax.numpy as jnp
from jax import lax
from jax.experimental import pallas as pl
from jax.experimental.pallas import tpu as pltpu
from jax.experimental.pallas import tpu_sc as plsc

F = 26
V1 = 100001
D = 32
L = 16
NW = 32
OCH = 4096


def _gather_kernel(B, idx_hbm, tab_hbm, out_hbm, idx_v, row_v, ob0, ob1,
                   *sems):
    obufs = (ob0, ob1)
    per_w = (F * D) // NW
    n_och = B // OCH
    wid = lax.axis_index("c") * 16 + lax.axis_index("s")
    p0 = wid * per_w

    def load_idx(f):
        pltpu.sync_copy(idx_hbm.at[f], idx_v)

        @plsc.parallel_loop(0, B // L, unroll=8)
        def _(i):
            sl = pl.ds(i * L, L)
            idx_v[sl] = jnp.minimum(jnp.maximum(idx_v[sl], 0), V1 - 1)

    def task(t, _):
        p = p0 + t
        f = lax.shift_right_logical(p, 5)
        d = lax.bitwise_and(p, D - 1)

        row_cp = pltpu.async_copy(tab_hbm.at[f, d], row_v, sems[2])

        @pl.when(jnp.logical_or(t == 0, d == 0))
        def _():
            load_idx(f)

        row_cp.wait()

        for c in range(n_och):
            buf = obufs[c % 2]
            prev = pltpu.make_async_copy(
                buf, out_hbm.at[0, 0, pl.ds(0, OCH)], sems[c % 2])
            if c >= 2:
                prev.wait()
            else:
                @pl.when(t > 0)
                def _():
                    prev.wait()

            @plsc.parallel_loop(0, OCH // L, unroll=8)
            def _(j):
                iv = idx_v[pl.ds(c * OCH + j * L, L)]
                buf[pl.ds(j * L, L)] = plsc.load_gather(row_v, [iv])
            pltpu.async_copy(
                buf, out_hbm.at[f, d, pl.ds(c * OCH, OCH)], sems[c % 2])
        return 0

    lax.fori_loop(0, per_w, task, 0)
    for k in range(2):
        pltpu.make_async_copy(
            obufs[k], out_hbm.at[0, 0, pl.ds(0, OCH)], sems[k]).wait()


def kernel(int_feats, tables):
    B, nf = int_feats.shape
    assert nf == F and tables.shape == (F, V1, D)
    assert (F * D) % NW == 0 and B % 256 == 0 and B % OCH == 0

    idx_t = int_feats.T
    tab_t = tables.transpose(0, 2, 1)

    mesh = plsc.VectorSubcoreMesh(core_axis_name="c", subcore_axis_name="s",
                                  num_cores=2, num_subcores=16)
    run = pl.kernel(
        functools.partial(_gather_kernel, B),
        out_type=jax.ShapeDtypeStruct((F, D, B), jnp.float32),
        mesh=mesh,
        scratch_types=(
            [pltpu.VMEM((B,), jnp.int32),
             pltpu.VMEM((V1,), jnp.float32),
             pltpu.VMEM((OCH,), jnp.float32),
             pltpu.VMEM((OCH,), jnp.float32)]
            + [pltpu.SemaphoreType.DMA] * 3
        ),
        compiler_params=pltpu.CompilerParams(needs_layout_passes=False),
    )
    out_t = run(idx_t, tab_t)
    return out_t.transpose(2, 0, 1)
